# baseline (speedup 1.0000x reference)
import functools

import jax
import jax.numpy as jnp
from jax.experimental import pallas as pl
from jax.experimental.pallas import tpu as pltpu

N_VARS = 16
N_TGT = 4
INPUT_DIM = 8
HIDDEN = 128
CLS_H = 64
NUM_CLASSES = 10
INV_DEG = 1.0 / 17.0


def _tree_sum(parts):
    while len(parts) > 1:
        parts = [parts[i] + parts[i + 1] for i in range(0, len(parts), 2)]
    return parts[0]


def _fwd_body(f_ref, w1_ref, w2_ref, b2_ref, c1w_ref, c1b_ref,
              c2w_ref, c2b_ref, hw1_ref, hb1_ref, hw2_ref, hb2_ref, out_ref):
    bb = f_ref.shape[1]

    fcat = f_ref[...].reshape(N_VARS * bb, INPUT_DIM + 1)
    h = jnp.maximum(jnp.dot(fcat, w1_ref[...], preferred_element_type=jnp.float32), 0.0)
    h2 = jnp.maximum(
        jnp.dot(h, w2_ref[...], preferred_element_type=jnp.float32) + b2_ref[...], 0.0)

    s = _tree_sum([h2[v * bb:(v + 1) * bb] for v in range(N_VARS)])
    y1 = jnp.dot(h2, c1w_ref[...], preferred_element_type=jnp.float32)
    z1 = jnp.dot(s, c1w_ref[...], preferred_element_type=jnp.float32) + c1b_ref[...]
    x1 = jnp.concatenate(
        [jnp.maximum(y1[v * bb:(v + 1) * bb] + z1, 0.0) for v in range(N_VARS)], axis=0)

    s1 = _tree_sum([x1[v * bb:(v + 1) * bb] for v in range(N_VARS)])
    y2 = jnp.dot(x1[:N_TGT * bb], c2w_ref[...], preferred_element_type=jnp.float32)
    z2 = jnp.dot(s1, c2w_ref[...], preferred_element_type=jnp.float32) + c2b_ref[...]
    x2 = jnp.concatenate(
        [jnp.maximum(y2[i * bb:(i + 1) * bb] + z2, 0.0) for i in range(N_TGT)], axis=0)

    for i in range(N_TGT):
        xi = x2[i * bb:(i + 1) * bb]
        hh = jnp.maximum(
            jnp.dot(xi, hw1_ref[i], preferred_element_type=jnp.float32) + hb1_ref[i], 0.0)
        out_ref[i] = jnp.dot(hh, hw2_ref[i], preferred_element_type=jnp.float32) + hb2_ref[i]


@functools.partial(jax.jit, static_argnames=("block_b",))
def _run(feats, w1, w2, b2, c1w, c1b, c2w, c2b, hw1, hb1, hw2, hb2,
         block_b=1024):
    b_total = feats.shape[1]
    grid = (b_total // block_b,)

    def full(shape):
        return pl.BlockSpec(shape, lambda i: (0,) * len(shape))

    out = pl.pallas_call(
        _fwd_body,
        grid=grid,
        in_specs=[
            pl.BlockSpec((N_VARS, block_b, INPUT_DIM + 1), lambda i: (0, i, 0)),
            full((INPUT_DIM + 1, HIDDEN)),
            full((HIDDEN, HIDDEN)), full((1, HIDDEN)),
            full((HIDDEN, HIDDEN)), full((1, HIDDEN)),
            full((HIDDEN, HIDDEN)), full((1, HIDDEN)),
            full((N_TGT, HIDDEN, CLS_H)), full((N_TGT, 1, CLS_H)),
            full((N_TGT, CLS_H, NUM_CLASSES)), full((N_TGT, 1, NUM_CLASSES)),
        ],
        out_specs=pl.BlockSpec((N_TGT, block_b, NUM_CLASSES), lambda i: (0, i, 0)),
        out_shape=jax.ShapeDtypeStruct((N_TGT, b_total, NUM_CLASSES), jnp.float32),
        compiler_params=pltpu.CompilerParams(
            dimension_semantics=("parallel",)),
    )(feats, w1, w2, b2, c1w, c1b, c2w, c2b, hw1, hb1, hw2, hb2)
    return out


def kernel(var_0_raw, var_1_raw, var_2_raw, var_3_raw, var_4_raw, var_5_raw,
           var_6_raw, var_7_raw, var_8_raw, var_9_raw, var_10_raw, var_11_raw,
           var_12_raw, var_13_raw, var_14_raw, var_15_raw, params):
    feats = jnp.stack(
        (var_0_raw, var_1_raw, var_2_raw, var_3_raw, var_4_raw, var_5_raw,
         var_6_raw, var_7_raw, var_8_raw, var_9_raw, var_10_raw, var_11_raw,
         var_12_raw, var_13_raw, var_14_raw, var_15_raw), axis=0)
    nv, b_total, _ = feats.shape
    feats = jnp.concatenate(
        [feats, jnp.ones((nv, b_total, 1), jnp.float32)], axis=-1)
    p = params
    w1 = jnp.concatenate([p["enc_W1"], p["enc_b1"].reshape(1, HIDDEN)], axis=0)
    targets = [f"var_{i}" for i in range(N_TGT)]
    hw1 = jnp.stack([p[f"cls_{t}_W1"] for t in targets], axis=0)
    hb1 = jnp.stack([p[f"cls_{t}_b1"].reshape(1, CLS_H) for t in targets], axis=0)
    hw2 = jnp.stack([p[f"cls_{t}_W2"] for t in targets], axis=0)
    hb2 = jnp.stack([p[f"cls_{t}_b2"].reshape(1, NUM_CLASSES) for t in targets], axis=0)
    return _run(
        feats, w1,
        p["enc_W2"], p["enc_b2"].reshape(1, HIDDEN),
        p["conv1_W"] * INV_DEG, p["conv1_b"].reshape(1, HIDDEN),
        p["conv2_W"] * INV_DEG, p["conv2_b"].reshape(1, HIDDEN),
        hw1, hb1, hw2, hb2)


# speedup vs baseline: 1.0392x; 1.0392x over previous
import functools

import jax
import jax.numpy as jnp
from jax.experimental import pallas as pl
from jax.experimental.pallas import tpu as pltpu

N_VARS = 16
N_TGT = 4
INPUT_DIM = 8
HIDDEN = 128
CLS_H = 64
NUM_CLASSES = 10
INV_DEG = 1.0 / 17.0


def _tree_sum(parts):
    while len(parts) > 1:
        parts = [parts[i] + parts[i + 1] for i in range(0, len(parts), 2)]
    return parts[0]


def _fwd_body(*refs):
    f_refs = refs[:N_VARS]
    (w1_ref, b1_ref, w2_ref, b2_ref, c1w_ref, c1b_ref, c2w_ref, c2b_ref) = \
        refs[N_VARS:N_VARS + 8]
    head_refs = refs[N_VARS + 8:N_VARS + 8 + 4 * N_TGT]
    out_ref = refs[-1]
    bb = f_refs[0].shape[0]

    fcat = jnp.concatenate([r[...] for r in f_refs], axis=0)
    h = jnp.maximum(
        jnp.dot(fcat, w1_ref[...], preferred_element_type=jnp.float32) + b1_ref[...], 0.0)
    h2 = jnp.maximum(
        jnp.dot(h, w2_ref[...], preferred_element_type=jnp.float32) + b2_ref[...], 0.0)

    c1w = c1w_ref[...] * INV_DEG
    s = _tree_sum([h2[v * bb:(v + 1) * bb] for v in range(N_VARS)])
    y1 = jnp.dot(h2, c1w, preferred_element_type=jnp.float32)
    z1 = jnp.dot(s, c1w, preferred_element_type=jnp.float32) + c1b_ref[...]
    x1 = jnp.concatenate(
        [jnp.maximum(y1[v * bb:(v + 1) * bb] + z1, 0.0) for v in range(N_VARS)], axis=0)

    c2w = c2w_ref[...] * INV_DEG
    s1 = _tree_sum([x1[v * bb:(v + 1) * bb] for v in range(N_VARS)])
    y2 = jnp.dot(x1[:N_TGT * bb], c2w, preferred_element_type=jnp.float32)
    z2 = jnp.dot(s1, c2w, preferred_element_type=jnp.float32) + c2b_ref[...]

    for i in range(N_TGT):
        hw1, hb1, hw2, hb2 = head_refs[4 * i:4 * i + 4]
        x2 = jnp.maximum(y2[i * bb:(i + 1) * bb] + z2, 0.0)
        hh = jnp.maximum(
            jnp.dot(x2, hw1[...], preferred_element_type=jnp.float32) + hb1[...], 0.0)
        out_ref[i] = jnp.dot(hh, hw2[...], preferred_element_type=jnp.float32) + hb2[...]


@functools.partial(jax.jit, static_argnames=("block_b",))
def _run(*args, block_b=1024):
    b_total = args[0].shape[0]
    grid = (b_total // block_b,)

    def full(shape):
        return pl.BlockSpec(shape, lambda i: (0,) * len(shape))

    var_spec = pl.BlockSpec((block_b, INPUT_DIM), lambda i: (i, 0))
    weight_specs = [
        full((INPUT_DIM, HIDDEN)), full((1, HIDDEN)),
        full((HIDDEN, HIDDEN)), full((1, HIDDEN)),
        full((HIDDEN, HIDDEN)), full((1, HIDDEN)),
        full((HIDDEN, HIDDEN)), full((1, HIDDEN)),
    ]
    head_specs = [
        full((HIDDEN, CLS_H)), full((1, CLS_H)),
        full((CLS_H, NUM_CLASSES)), full((1, NUM_CLASSES)),
    ] * N_TGT

    out = pl.pallas_call(
        _fwd_body,
        grid=grid,
        in_specs=[var_spec] * N_VARS + weight_specs + head_specs,
        out_specs=pl.BlockSpec((N_TGT, block_b, NUM_CLASSES), lambda i: (0, i, 0)),
        out_shape=jax.ShapeDtypeStruct((N_TGT, b_total, NUM_CLASSES), jnp.float32),
        compiler_params=pltpu.CompilerParams(
            dimension_semantics=("parallel",)),
    )(*args)
    return out


def kernel(var_0_raw, var_1_raw, var_2_raw, var_3_raw, var_4_raw, var_5_raw,
           var_6_raw, var_7_raw, var_8_raw, var_9_raw, var_10_raw, var_11_raw,
           var_12_raw, var_13_raw, var_14_raw, var_15_raw, params):
    p = params
    head_args = []
    for i in range(N_TGT):
        t = f"var_{i}"
        head_args += [
            p[f"cls_{t}_W1"], p[f"cls_{t}_b1"].reshape(1, CLS_H),
            p[f"cls_{t}_W2"], p[f"cls_{t}_b2"].reshape(1, NUM_CLASSES),
        ]
    return _run(
        var_0_raw, var_1_raw, var_2_raw, var_3_raw, var_4_raw, var_5_raw,
        var_6_raw, var_7_raw, var_8_raw, var_9_raw, var_10_raw, var_11_raw,
        var_12_raw, var_13_raw, var_14_raw, var_15_raw,
        p["enc_W1"], p["enc_b1"].reshape(1, HIDDEN),
        p["enc_W2"], p["enc_b2"].reshape(1, HIDDEN),
        p["conv1_W"], p["conv1_b"].reshape(1, HIDDEN),
        p["conv2_W"], p["conv2_b"].reshape(1, HIDDEN),
        *head_args)
